# baseline (speedup 1.0000x reference)
import functools

import jax
import jax.numpy as jnp
from jax import lax
from jax.experimental import pallas as pl
from jax.experimental.pallas import tpu as pltpu
from jax.experimental.pallas import tpu_sc as plsc

N = 10000
E = 160000
F_IN = 256
H = 64
NP = 10240
NW = 32
EPW = E // NW
NPT = NP // NW
GSZ = 64
EPWP = 5120
NG = EPWP // GSZ
TCH = 8
ROWS_PER_SUB = NP // 16

_mesh = plsc.VectorSubcoreMesh(core_axis_name="c", subcore_axis_name="s")

_SC_PARAMS = pltpu.CompilerParams(needs_layout_passes=False)

_HI = jax.lax.Precision.HIGHEST


def _wid():
    return lax.axis_index("s") * 2 + lax.axis_index("c")


def _iota16():
    return lax.broadcasted_iota(jnp.int32, (16,), 0)


def _dup_ranks(d16):
    iota = _iota16()
    rank = jnp.zeros((16,), jnp.int32)
    later = jnp.zeros((16,), jnp.int32)
    for k in range(16):
        eq = d16 == jnp.full((16,), d16[k])
        rank = rank + jnp.where(eq & (iota > k), 1, 0)
        later = later + jnp.where(eq & (iota < k), 1, 0)
    return rank, later


@functools.partial(
    pl.kernel,
    out_type=jax.ShapeDtypeStruct((NW, NP), jnp.int32),
    mesh=_mesh,
    compiler_params=_SC_PARAMS,
    scratch_types=[
        pltpu.VMEM((EPWP,), jnp.int32),
        pltpu.VMEM((NP,), jnp.int32),
    ],
)
def _sc_hist(dst_hbm, out_hbm, dst_v, cnt_v):
    w = _wid()

    def pad_body(i, carry):
        dst_v[pl.ds(EPWP - 128 + i * 16, 16)] = jnp.full((16,), NP - 1, jnp.int32)
        return carry

    lax.fori_loop(0, 8, pad_body, 0)
    pltpu.sync_copy(dst_hbm.at[pl.ds(w * EPW, EPW)], dst_v.at[pl.ds(0, EPW)])

    def zero_body(i, carry):
        cnt_v[pl.ds(i * 16, 16)] = jnp.zeros((16,), jnp.int32)
        return carry

    lax.fori_loop(0, NP // 16, zero_body, 0)

    def hist_body(g, carry):
        d16 = dst_v[pl.ds(g * 16, 16)]
        rank, later = _dup_ranks(d16)
        cur = plsc.load_gather(cnt_v, [d16])
        valid = (_iota16() + jnp.full((16,), g * 16)) < EPW
        plsc.store_scatter(cnt_v, [d16], cur + rank + 1,
                           mask=valid & (later == 0))
        return carry

    lax.fori_loop(0, (EPW + 15) // 16, hist_body, 0)
    pltpu.sync_copy(cnt_v, out_hbm.at[w])


def _tc_offsets_body(cnt_ref, counts_ref, starts_ref, invc_ref, off_ref, maxd_ref):
    cnt = cnt_ref[...].astype(jnp.float32)
    counts = jnp.sum(cnt, axis=0)

    r0 = lax.broadcasted_iota(jnp.int32, (80, 80), 0)
    c0 = lax.broadcasted_iota(jnp.int32, (80, 80), 1)
    lt80 = (c0 < r0).astype(jnp.float32)
    r1 = lax.broadcasted_iota(jnp.int32, (128, 128), 0)
    c1 = lax.broadcasted_iota(jnp.int32, (128, 128), 1)
    ut128 = (r1 < c1).astype(jnp.float32)

    s_rows = jax.lax.dot_general(lt80, counts, (((1,), (0,)), ((), ())),
                                 precision=_HI)
    term1 = jnp.sum(s_rows, axis=1, keepdims=True)
    term2 = jax.lax.dot_general(counts, ut128, (((1,), (0,)), ((), ())),
                                precision=_HI)
    starts = term1 + term2

    counts_ref[...] = counts.astype(jnp.int32)
    starts_ref[...] = starts.astype(jnp.int32)
    invc_ref[...] = 1.0 / jnp.maximum(counts, 1.0)
    maxd_ref[...] = jnp.full((1, 1), jnp.max(counts), jnp.float32).astype(jnp.int32)

    acc = jnp.zeros((80, 128), jnp.float32)
    for w in range(NW):
        off_ref[w] = (acc + starts).astype(jnp.int32)
        acc = acc + cnt[w]


def _tc_offsets(cnt3):
    return pl.pallas_call(
        _tc_offsets_body,
        out_shape=[
            jax.ShapeDtypeStruct((80, 128), jnp.int32),
            jax.ShapeDtypeStruct((80, 128), jnp.int32),
            jax.ShapeDtypeStruct((80, 128), jnp.float32),
            jax.ShapeDtypeStruct((NW, 80, 128), jnp.int32),
            jax.ShapeDtypeStruct((1, 1), jnp.int32),
        ],
    )(cnt3)


def _tc_proj_body(x_ref, wl_ref, wr_ref, b1_ref, yl_ref, xr_ref):
    xb = x_ref[...]
    yl = jax.lax.dot_general(xb, wl_ref[...], (((1,), (0,)), ((), ())),
                             precision=_HI)
    yl_ref[...] = jnp.concatenate([yl, jnp.zeros_like(yl)], axis=1)
    xr_ref[...] = jax.lax.dot_general(xb, wr_ref[...], (((1,), (0,)), ((), ())),
                                      precision=_HI) + b1_ref[...]


def _tc_proj(xp, wlT, wrT, b1):
    nb = 8
    blk = NP // nb
    return pl.pallas_call(
        _tc_proj_body,
        grid=(nb,),
        in_specs=[
            pl.BlockSpec((blk, F_IN), lambda i: (i, 0)),
            pl.BlockSpec((F_IN, H), lambda i: (0, 0)),
            pl.BlockSpec((F_IN, H), lambda i: (0, 0)),
            pl.BlockSpec((1, H), lambda i: (0, 0)),
        ],
        out_specs=[
            pl.BlockSpec((blk, 2 * H), lambda i: (i, 0)),
            pl.BlockSpec((blk, H), lambda i: (i, 0)),
        ],
        out_shape=[
            jax.ShapeDtypeStruct((NP, 2 * H), jnp.float32),
            jax.ShapeDtypeStruct((NP, H), jnp.float32),
        ],
    )(xp, wlT, wrT, b1)


@functools.partial(
    pl.kernel,
    out_type=[
        jax.ShapeDtypeStruct((E + 2 * NW,), jnp.int32),
        jax.ShapeDtypeStruct((2, NP, 2 * H), jnp.float32),
    ],
    mesh=_mesh,
    compiler_params=_SC_PARAMS,
    scratch_types=[
        pltpu.VMEM((EPWP,), jnp.int32),
        pltpu.VMEM((EPWP,), jnp.int32),
        pltpu.VMEM((NP,), jnp.int32),
        pltpu.VMEM((NG, GSZ), jnp.int32),
        pltpu.VMEM((GSZ, 2 * H), jnp.float32),
        pltpu.VMEM((GSZ, 2 * H), jnp.float32),
        pltpu.VMEM_SHARED((NP, 2 * H), jnp.float32),
        pltpu.SemaphoreType.DMA,
        pltpu.SemaphoreType.DMA,
        pltpu.SemaphoreType.DMA,
        pltpu.SemaphoreType.DMA,
        pltpu.SemaphoreType.DMA,
    ],
)
def _sc_csr(dst_hbm, src_hbm, off_hbm, yl_hbm, srt_hbm, spar_hbm,
            src_v, dst_v, off_v, slot_v,
            rows0_v, rows1_v, s_sh,
            sem_s, semg0, semg1, sema0, sema1):
    c = lax.axis_index("c")
    s = lax.axis_index("s")
    w = _wid()

    def pad_body(i, carry):
        src_v[pl.ds(EPWP - 128 + i * 16, 16)] = jnp.full((16,), NP - 1, jnp.int32)
        dst_v[pl.ds(EPWP - 128 + i * 16, 16)] = jnp.full((16,), NP - 1, jnp.int32)
        return carry

    lax.fori_loop(0, 8, pad_body, 0)
    pltpu.sync_copy(src_hbm.at[pl.ds(w * EPW, EPW)], src_v.at[pl.ds(0, EPW)])
    pltpu.sync_copy(dst_hbm.at[pl.ds(w * EPW, EPW)], dst_v.at[pl.ds(0, EPW)])
    pltpu.sync_copy(off_hbm.at[w], off_v)

    def pre_body(i, carry):
        slot_v[lax.shift_right_logical(i, 2), pl.ds((i & 3) * 16, 16)] = (
            jnp.full((16,), E + 2 * w, jnp.int32))
        return carry

    lax.fori_loop(0, NG * 4, pre_body, 0)

    def zrow_body(i, carry):
        rows0_v[lax.shift_right_logical(i, 2), pl.ds((i & 3) * 16, 16)] = (
            jnp.zeros((16,), jnp.float32))
        return carry

    lax.fori_loop(0, GSZ * 4, zrow_body, 0)
    zc = [pltpu.async_copy(rows0_v,
                           s_sh.at[pl.ds(s * ROWS_PER_SUB + q * GSZ, GSZ)],
                           sem_s)
          for q in range(ROWS_PER_SUB // GSZ)]
    for cp in zc:
        cp.wait()
    plsc.subcore_barrier()

    def slot_body(i, carry):
        d16 = dst_v[pl.ds(i * 16, 16)]
        rank, later = _dup_ranks(d16)
        cur = plsc.load_gather(off_v, [d16])
        valid = (_iota16() + jnp.full((16,), i * 16)) < EPW
        plsc.store_scatter(off_v, [d16], cur + rank + 1,
                           mask=valid & (later == 0))
        g = lax.shift_right_logical(i, 2)
        r = lax.bitwise_and(i, 3) * 16
        slot_v[g, pl.ds(r, 16)] = cur + rank
        return carry

    lax.fori_loop(0, (EPW + 15) // 16, slot_body, 0)

    for g0 in range(0, NG, 10):
        sc_c = [pltpu.async_copy(src_v.at[pl.ds(g * GSZ, GSZ)],
                                 srt_hbm.at[slot_v.at[g]], sem_s)
                for g in range(g0, min(g0 + 10, NG))]
        for cp in sc_c:
            cp.wait()

    bufs = (rows0_v, rows1_v)
    gsems = (semg0, semg1)
    asems = (sema0, sema1)
    pend_g = [None, None]
    pend_a = [None, None]

    def fireg(g):
        b = g & 1
        pend_g[b] = pltpu.async_copy(
            yl_hbm.at[src_v.at[pl.ds(g * GSZ, GSZ)]], bufs[b], gsems[b])

    fireg(0)
    for g in range(NG):
        b = g & 1
        nb_ = (g + 1) & 1
        if g + 1 < NG:
            if pend_a[nb_] is not None:
                pend_a[nb_].wait()
                pend_a[nb_] = None
            fireg(g + 1)
        pend_g[b].wait()
        pend_a[b] = pltpu.async_copy(
            bufs[b], s_sh.at[dst_v.at[pl.ds(g * GSZ, GSZ)]],
            asems[b], add=True)
    for b in (0, 1):
        if pend_a[b] is not None:
            pend_a[b].wait()
    plsc.subcore_barrier()

    base = s * ROWS_PER_SUB
    pltpu.sync_copy(s_sh.at[pl.ds(base, ROWS_PER_SUB)],
                    spar_hbm.at[c, pl.ds(base, ROWS_PER_SUB)])


def _tc_x1_body(s_ref, invc_ref, xr_ref, x1_ref):
    i = pl.program_id(0)
    blk = x1_ref.shape[0]
    srow = (s_ref[0, :, :H] + s_ref[1, :, :H]) * invc_ref[...]
    h1 = srow + xr_ref[...]
    x1 = jnp.where(h1 > 0, h1, jnp.exp(jnp.minimum(h1, 0.0)) - 1.0)
    rows = i * blk + lax.broadcasted_iota(jnp.int32, (blk, 1), 0)
    x1 = jnp.where(rows < N, x1, 0.0)
    x1_ref[...] = jnp.concatenate([x1, jnp.zeros_like(x1)], axis=1)


def _tc_x1(spar, invc_col, xr):
    nb = 8
    blk = NP // nb
    return pl.pallas_call(
        _tc_x1_body,
        grid=(nb,),
        in_specs=[
            pl.BlockSpec((2, blk, 2 * H), lambda i: (0, i, 0)),
            pl.BlockSpec((blk, 1), lambda i: (i, 0)),
            pl.BlockSpec((blk, H), lambda i: (i, 0)),
        ],
        out_specs=pl.BlockSpec((blk, 2 * H), lambda i: (i, 0)),
        out_shape=jax.ShapeDtypeStruct((NP, 2 * H), jnp.float32),
    )(spar, invc_col, xr)


NSV = NPT // 16
PALL = TCH * NPT


@functools.partial(
    pl.kernel,
    out_type=jax.ShapeDtypeStruct((TCH, NP, 2 * H), jnp.float32),
    mesh=_mesh,
    compiler_params=_SC_PARAMS,
    scratch_types=[
        pltpu.VMEM((16,), jnp.int32),
        pltpu.VMEM((NPT,), jnp.int32),
        pltpu.VMEM((PALL,), jnp.int32),
        pltpu.VMEM((PALL,), jnp.int32),
        pltpu.VMEM((NPT, 2 * H), jnp.float32),
        pltpu.VMEM((NPT, 2 * H), jnp.float32),
        pltpu.VMEM((NPT, 2 * H), jnp.float32),
        pltpu.SemaphoreType.DMA,
        pltpu.SemaphoreType.DMA,
        pltpu.SemaphoreType.DMA,
        pltpu.SemaphoreType.DMA,
        pltpu.SemaphoreType.DMA,
        pltpu.SemaphoreType.DMA,
    ],
)
def _sc_gather(t0_hbm, st_hbm, srt_hbm, x1_hbm, seq_hbm,
               t0_v, st_v, p_v, sid_v, rows0_v, rows1_v, rows2_v,
               semg0, semg1, semg2, semw0, semw1, semw2):
    w = _wid()
    base = w * NPT
    pltpu.sync_copy(t0_hbm, t0_v)
    pltpu.sync_copy(st_hbm.at[pl.ds(base, NPT)], st_v)
    t0 = t0_v[...][0]

    for j in range(TCH):
        def pos_body(v, carry, j=j):
            s16 = st_v[pl.ds(v * 16, 16)]
            p_v[pl.ds(j * NPT + v * 16, 16)] = jnp.minimum(
                s16 + jnp.full((16,), t0 + j), E - 1)
            return carry

        lax.fori_loop(0, NSV, pos_body, 0)

    pltpu.sync_copy(srt_hbm.at[p_v], sid_v)

    bufs = (rows0_v, rows1_v, rows2_v)
    gsems = (semg0, semg1, semg2)
    wsems = (semw0, semw1, semw2)
    pend_g = [None, None, None]
    pend_w = [None, None, None]

    def fire(j):
        b = j % 3
        pend_g[b] = pltpu.async_copy(
            x1_hbm.at[sid_v.at[pl.ds(j * NPT, NPT)]], bufs[b], gsems[b])

    for j in range(min(2, TCH)):
        fire(j)
    for j in range(TCH):
        b = j % 3
        nxt = j + 2
        if nxt < TCH:
            nb_ = nxt % 3
            if pend_w[nb_] is not None:
                pend_w[nb_].wait()
                pend_w[nb_] = None
            fire(nxt)
        pend_g[b].wait()
        pend_w[b] = pltpu.async_copy(
            bufs[b], seq_hbm.at[j, pl.ds(base, NPT)], wsems[b])
    for b in range(3):
        if pend_w[b] is not None:
            pend_w[b].wait()


def _tc_lstm_body(sc_ref, cnt_ref, seq_ref, h_ref, c_ref, w_ref, b_ref,
                  ho_ref, co_ref):
    ho_ref[...] = h_ref[...]
    co_ref[...] = c_ref[...]
    rem = sc_ref[0]
    t0 = sc_ref[1]
    cnt = cnt_ref[...]
    for j in range(TCH):
        @pl.when(j < rem)
        def _():
            tf = (t0 + j).astype(jnp.float32)
            xt = jnp.where(cnt > tf, seq_ref[j][:, :H], 0.0)
            hh = ho_ref[...]
            xh = jnp.concatenate([xt, hh], axis=1)
            g = jax.lax.dot_general(xh, w_ref[...], (((1,), (0,)), ((), ())),
                                    precision=_HI) + b_ref[...]
            gi = g[:, 0 * H:1 * H]
            gf = g[:, 1 * H:2 * H]
            gg = g[:, 2 * H:3 * H]
            go = g[:, 3 * H:4 * H]
            cn = jax.nn.sigmoid(gf) * co_ref[...] + jax.nn.sigmoid(gi) * jnp.tanh(gg)
            co_ref[...] = cn
            ho_ref[...] = jax.nn.sigmoid(go) * jnp.tanh(cn)


def _tc_lstm(scal, cntf, seq, h, c, wcat, bias):
    nb = 8
    blk = NP // nb
    return pl.pallas_call(
        _tc_lstm_body,
        grid=(nb,),
        in_specs=[
            pl.BlockSpec(memory_space=pltpu.SMEM),
            pl.BlockSpec((blk, 1), lambda i: (i, 0)),
            pl.BlockSpec((TCH, blk, 2 * H), lambda i: (0, i, 0)),
            pl.BlockSpec((blk, H), lambda i: (i, 0)),
            pl.BlockSpec((blk, H), lambda i: (i, 0)),
            pl.BlockSpec((2 * H, 4 * H), lambda i: (0, 0)),
            pl.BlockSpec((1, 4 * H), lambda i: (0, 0)),
        ],
        out_specs=[
            pl.BlockSpec((blk, H), lambda i: (i, 0)),
            pl.BlockSpec((blk, H), lambda i: (i, 0)),
        ],
        out_shape=[
            jax.ShapeDtypeStruct((NP, H), jnp.float32),
            jax.ShapeDtypeStruct((NP, H), jnp.float32),
        ],
    )(scal, cntf, seq, h, c, wcat, bias)


def _tc_out_body(h_ref, x1_ref, wl_ref, wr_ref, b2_ref, o_ref):
    o = (jax.lax.dot_general(h_ref[...], wl_ref[...], (((1,), (0,)), ((), ())),
                             precision=_HI)
         + jax.lax.dot_general(x1_ref[...][:, :H], wr_ref[...], (((1,), (0,)), ((), ())),
                               precision=_HI)
         + b2_ref[...])
    m = jnp.max(o, axis=1, keepdims=True)
    z = o - m
    o_ref[...] = z - jnp.log(jnp.sum(jnp.exp(z), axis=1, keepdims=True))


def _tc_out(hT, x1p, wlT, wrT, b2):
    nb = 8
    blk = NP // nb
    return pl.pallas_call(
        _tc_out_body,
        grid=(nb,),
        in_specs=[
            pl.BlockSpec((blk, H), lambda i: (i, 0)),
            pl.BlockSpec((blk, 2 * H), lambda i: (i, 0)),
            pl.BlockSpec((H, H), lambda i: (0, 0)),
            pl.BlockSpec((H, H), lambda i: (0, 0)),
            pl.BlockSpec((1, H), lambda i: (0, 0)),
        ],
        out_specs=pl.BlockSpec((blk, H), lambda i: (i, 0)),
        out_shape=jax.ShapeDtypeStruct((NP, H), jnp.float32),
    )(hT, x1p, wlT, wrT, b2)


@jax.jit
def kernel(x, edge_index, W1_l, b1_l, W1_r, W_ih, W_hh, b_ih, b_hh, W2_l, b2_l, W2_r):
    src = edge_index[0]
    dst = edge_index[1]
    xp = jnp.pad(x, ((0, NP - N), (0, 0)))

    cnt_all = _sc_hist(dst)
    counts2, starts2, invc2, off3, maxd = _tc_offsets(cnt_all.reshape(NW, 80, 128))
    ylp, xr = _tc_proj(xp, W1_l.T, W1_r.T, b1_l.reshape(1, H))

    srt, spar = _sc_csr(dst, src, off3.reshape(NW, NP), ylp)
    x1p = _tc_x1(spar, invc2.reshape(NP, 1), xr)

    counts_f = counts2.reshape(NP)
    starts_f = starts2.reshape(NP)
    maxdeg = maxd[0, 0]

    wcat = jnp.concatenate([W_ih.T, W_hh.T], axis=0)
    bias = (b_ih + b_hh).reshape(1, 4 * H)

    h0 = jnp.zeros((NP, H), jnp.float32)
    c0 = jnp.zeros((NP, H), jnp.float32)

    def cond(carry):
        t0, _, _ = carry
        return t0 < maxdeg

    cnt_colf = counts_f.reshape(NP, 1).astype(jnp.float32)

    def body(carry):
        t0, h, c = carry
        t0a = jnp.full((16,), t0, jnp.int32)
        seq = _sc_gather(t0a, starts_f, srt, x1p)
        scal = jnp.stack([maxdeg - t0, t0])
        h, c = _tc_lstm(scal, cnt_colf, seq, h, c, wcat, bias)
        return t0 + TCH, h, c

    _, hT, _ = lax.while_loop(cond, body, (jnp.int32(0), h0, c0))

    out = _tc_out(hT, x1p, W2_l.T, W2_r.T, b2_l.reshape(1, H))
    return out[:N]


# speedup vs baseline: 1.0011x; 1.0011x over previous
import functools

import jax
import jax.numpy as jnp
from jax import lax
from jax.experimental import pallas as pl
from jax.experimental.pallas import tpu as pltpu
from jax.experimental.pallas import tpu_sc as plsc

N = 10000
E = 160000
F_IN = 256
H = 64
NP = 10240
NW = 32
EPW = E // NW
NPT = NP // NW
GSZ = 64
EPWP = 5120
NG = EPWP // GSZ
TCH = 8
ROWS_PER_SUB = NP // 16

_mesh = plsc.VectorSubcoreMesh(core_axis_name="c", subcore_axis_name="s")

_SC_PARAMS = pltpu.CompilerParams(needs_layout_passes=False)

_HI = jax.lax.Precision.HIGHEST


def _wid():
    return lax.axis_index("s") * 2 + lax.axis_index("c")


def _iota16():
    return lax.broadcasted_iota(jnp.int32, (16,), 0)


def _dup_ranks(d16):
    iota = _iota16()
    rank = jnp.zeros((16,), jnp.int32)
    later = jnp.zeros((16,), jnp.int32)
    for k in range(16):
        eq = d16 == jnp.full((16,), d16[k])
        rank = rank + jnp.where(eq & (iota > k), 1, 0)
        later = later + jnp.where(eq & (iota < k), 1, 0)
    return rank, later


@functools.partial(
    pl.kernel,
    out_type=jax.ShapeDtypeStruct((NW, NP), jnp.int32),
    mesh=_mesh,
    compiler_params=_SC_PARAMS,
    scratch_types=[
        pltpu.VMEM((EPWP,), jnp.int32),
        pltpu.VMEM((NP,), jnp.int32),
    ],
)
def _sc_hist(dst_hbm, out_hbm, dst_v, cnt_v):
    w = _wid()

    def pad_body(i, carry):
        dst_v[pl.ds(EPWP - 128 + i * 16, 16)] = jnp.full((16,), NP - 1, jnp.int32)
        return carry

    lax.fori_loop(0, 8, pad_body, 0)
    pltpu.sync_copy(dst_hbm.at[pl.ds(w * EPW, EPW)], dst_v.at[pl.ds(0, EPW)])

    def zero_body(i, carry):
        cnt_v[pl.ds(i * 16, 16)] = jnp.zeros((16,), jnp.int32)
        return carry

    lax.fori_loop(0, NP // 16, zero_body, 0)

    def hist_body(g, carry):
        d16 = dst_v[pl.ds(g * 16, 16)]
        rank, later = _dup_ranks(d16)
        cur = plsc.load_gather(cnt_v, [d16])
        valid = (_iota16() + jnp.full((16,), g * 16)) < EPW
        plsc.store_scatter(cnt_v, [d16], cur + rank + 1,
                           mask=valid & (later == 0))
        return carry

    lax.fori_loop(0, (EPW + 15) // 16, hist_body, 0)
    pltpu.sync_copy(cnt_v, out_hbm.at[w])


def _tc_offsets_body(cnt_ref, counts_ref, starts_ref, invc_ref, off_ref, maxd_ref):
    cnt = cnt_ref[...].astype(jnp.float32)
    counts = jnp.sum(cnt, axis=0)

    r0 = lax.broadcasted_iota(jnp.int32, (80, 80), 0)
    c0 = lax.broadcasted_iota(jnp.int32, (80, 80), 1)
    lt80 = (c0 < r0).astype(jnp.float32)
    r1 = lax.broadcasted_iota(jnp.int32, (128, 128), 0)
    c1 = lax.broadcasted_iota(jnp.int32, (128, 128), 1)
    ut128 = (r1 < c1).astype(jnp.float32)

    s_rows = jax.lax.dot_general(lt80, counts, (((1,), (0,)), ((), ())),
                                 precision=_HI)
    term1 = jnp.sum(s_rows, axis=1, keepdims=True)
    term2 = jax.lax.dot_general(counts, ut128, (((1,), (0,)), ((), ())),
                                precision=_HI)
    starts = term1 + term2

    counts_ref[...] = counts.astype(jnp.int32)
    starts_ref[...] = starts.astype(jnp.int32)
    invc_ref[...] = 1.0 / jnp.maximum(counts, 1.0)
    maxd_ref[...] = jnp.full((1, 1), jnp.max(counts), jnp.float32).astype(jnp.int32)

    acc = jnp.zeros((80, 128), jnp.float32)
    for w in range(NW):
        off_ref[w] = (acc + starts).astype(jnp.int32)
        acc = acc + cnt[w]


def _tc_offsets(cnt3):
    return pl.pallas_call(
        _tc_offsets_body,
        out_shape=[
            jax.ShapeDtypeStruct((80, 128), jnp.int32),
            jax.ShapeDtypeStruct((80, 128), jnp.int32),
            jax.ShapeDtypeStruct((80, 128), jnp.float32),
            jax.ShapeDtypeStruct((NW, 80, 128), jnp.int32),
            jax.ShapeDtypeStruct((1, 1), jnp.int32),
        ],
    )(cnt3)


def _tc_proj_body(x_ref, wl_ref, wr_ref, b1_ref, yl_ref, xr_ref):
    xb = x_ref[...]
    yl = jax.lax.dot_general(xb, wl_ref[...], (((1,), (0,)), ((), ())),
                             precision=_HI)
    yl_ref[...] = jnp.concatenate([yl, jnp.zeros_like(yl)], axis=1)
    xr_ref[...] = jax.lax.dot_general(xb, wr_ref[...], (((1,), (0,)), ((), ())),
                                      precision=_HI) + b1_ref[...]


def _tc_proj(xp, wlT, wrT, b1):
    nb = 8
    blk = NP // nb
    return pl.pallas_call(
        _tc_proj_body,
        grid=(nb,),
        in_specs=[
            pl.BlockSpec((blk, F_IN), lambda i: (i, 0)),
            pl.BlockSpec((F_IN, H), lambda i: (0, 0)),
            pl.BlockSpec((F_IN, H), lambda i: (0, 0)),
            pl.BlockSpec((1, H), lambda i: (0, 0)),
        ],
        out_specs=[
            pl.BlockSpec((blk, 2 * H), lambda i: (i, 0)),
            pl.BlockSpec((blk, H), lambda i: (i, 0)),
        ],
        out_shape=[
            jax.ShapeDtypeStruct((NP, 2 * H), jnp.float32),
            jax.ShapeDtypeStruct((NP, H), jnp.float32),
        ],
    )(xp, wlT, wrT, b1)


@functools.partial(
    pl.kernel,
    out_type=[
        jax.ShapeDtypeStruct((E + 2 * NW,), jnp.int32),
        jax.ShapeDtypeStruct((2, NP, 2 * H), jnp.float32),
    ],
    mesh=_mesh,
    compiler_params=_SC_PARAMS,
    scratch_types=[
        pltpu.VMEM((EPWP,), jnp.int32),
        pltpu.VMEM((EPWP,), jnp.int32),
        pltpu.VMEM((NP,), jnp.int32),
        pltpu.VMEM((NG, GSZ), jnp.int32),
        pltpu.VMEM((GSZ, 2 * H), jnp.float32),
        pltpu.VMEM((GSZ, 2 * H), jnp.float32),
        pltpu.VMEM_SHARED((NP, 2 * H), jnp.float32),
        pltpu.SemaphoreType.DMA,
        pltpu.SemaphoreType.DMA,
        pltpu.SemaphoreType.DMA,
        pltpu.SemaphoreType.DMA,
        pltpu.SemaphoreType.DMA,
    ],
)
def _sc_csr(dst_hbm, src_hbm, off_hbm, yl_hbm, srt_hbm, spar_hbm,
            src_v, dst_v, off_v, slot_v,
            rows0_v, rows1_v, s_sh,
            sem_s, semg0, semg1, sema0, sema1):
    c = lax.axis_index("c")
    s = lax.axis_index("s")
    w = _wid()

    def pad_body(i, carry):
        src_v[pl.ds(EPWP - 128 + i * 16, 16)] = jnp.full((16,), NP - 1, jnp.int32)
        dst_v[pl.ds(EPWP - 128 + i * 16, 16)] = jnp.full((16,), NP - 1, jnp.int32)
        return carry

    lax.fori_loop(0, 8, pad_body, 0)
    pltpu.sync_copy(src_hbm.at[pl.ds(w * EPW, EPW)], src_v.at[pl.ds(0, EPW)])
    pltpu.sync_copy(dst_hbm.at[pl.ds(w * EPW, EPW)], dst_v.at[pl.ds(0, EPW)])
    pltpu.sync_copy(off_hbm.at[w], off_v)

    def pre_body(i, carry):
        slot_v[lax.shift_right_logical(i, 2), pl.ds((i & 3) * 16, 16)] = (
            jnp.full((16,), E + 2 * w, jnp.int32))
        return carry

    lax.fori_loop(0, NG * 4, pre_body, 0)

    def zrow_body(i, carry):
        rows0_v[lax.shift_right_logical(i, 2), pl.ds((i & 3) * 16, 16)] = (
            jnp.zeros((16,), jnp.float32))
        return carry

    lax.fori_loop(0, GSZ * 4, zrow_body, 0)
    zc = [pltpu.async_copy(rows0_v,
                           s_sh.at[pl.ds(s * ROWS_PER_SUB + q * GSZ, GSZ)],
                           sem_s)
          for q in range(ROWS_PER_SUB // GSZ)]
    for cp in zc:
        cp.wait()
    plsc.subcore_barrier()

    def slot_body(i, carry):
        d16 = dst_v[pl.ds(i * 16, 16)]
        rank, later = _dup_ranks(d16)
        cur = plsc.load_gather(off_v, [d16])
        valid = (_iota16() + jnp.full((16,), i * 16)) < EPW
        plsc.store_scatter(off_v, [d16], cur + rank + 1,
                           mask=valid & (later == 0))
        g = lax.shift_right_logical(i, 2)
        r = lax.bitwise_and(i, 3) * 16
        slot_v[g, pl.ds(r, 16)] = cur + rank
        return carry

    lax.fori_loop(0, (EPW + 15) // 16, slot_body, 0)

    prev_wave = []
    for g0 in range(0, NG, 10):
        wave = [pltpu.async_copy(src_v.at[pl.ds(g * GSZ, GSZ)],
                                 srt_hbm.at[slot_v.at[g]], sem_s)
                for g in range(g0, min(g0 + 10, NG))]
        for cp in prev_wave:
            cp.wait()
        prev_wave = wave
    for cp in prev_wave:
        cp.wait()

    bufs = (rows0_v, rows1_v)
    gsems = (semg0, semg1)
    asems = (sema0, sema1)
    pend_g = [None, None]
    pend_a = [None, None]

    def fireg(g):
        b = g & 1
        pend_g[b] = pltpu.async_copy(
            yl_hbm.at[src_v.at[pl.ds(g * GSZ, GSZ)]], bufs[b], gsems[b])

    fireg(0)
    for g in range(NG):
        b = g & 1
        nb_ = (g + 1) & 1
        if g + 1 < NG:
            if pend_a[nb_] is not None:
                pend_a[nb_].wait()
                pend_a[nb_] = None
            fireg(g + 1)
        pend_g[b].wait()
        pend_a[b] = pltpu.async_copy(
            bufs[b], s_sh.at[dst_v.at[pl.ds(g * GSZ, GSZ)]],
            asems[b], add=True)
    for b in (0, 1):
        if pend_a[b] is not None:
            pend_a[b].wait()
    plsc.subcore_barrier()

    base = s * ROWS_PER_SUB
    pltpu.sync_copy(s_sh.at[pl.ds(base, ROWS_PER_SUB)],
                    spar_hbm.at[c, pl.ds(base, ROWS_PER_SUB)])


def _tc_x1_body(s_ref, invc_ref, xr_ref, x1_ref):
    i = pl.program_id(0)
    blk = x1_ref.shape[0]
    srow = (s_ref[0, :, :H] + s_ref[1, :, :H]) * invc_ref[...]
    h1 = srow + xr_ref[...]
    x1 = jnp.where(h1 > 0, h1, jnp.exp(jnp.minimum(h1, 0.0)) - 1.0)
    rows = i * blk + lax.broadcasted_iota(jnp.int32, (blk, 1), 0)
    x1 = jnp.where(rows < N, x1, 0.0)
    x1_ref[...] = jnp.concatenate([x1, jnp.zeros_like(x1)], axis=1)


def _tc_x1(spar, invc_col, xr):
    nb = 8
    blk = NP // nb
    return pl.pallas_call(
        _tc_x1_body,
        grid=(nb,),
        in_specs=[
            pl.BlockSpec((2, blk, 2 * H), lambda i: (0, i, 0)),
            pl.BlockSpec((blk, 1), lambda i: (i, 0)),
            pl.BlockSpec((blk, H), lambda i: (i, 0)),
        ],
        out_specs=pl.BlockSpec((blk, 2 * H), lambda i: (i, 0)),
        out_shape=jax.ShapeDtypeStruct((NP, 2 * H), jnp.float32),
    )(spar, invc_col, xr)


NSV = NPT // 16
PALL = TCH * NPT


@functools.partial(
    pl.kernel,
    out_type=jax.ShapeDtypeStruct((TCH, NP, 2 * H), jnp.float32),
    mesh=_mesh,
    compiler_params=_SC_PARAMS,
    scratch_types=[
        pltpu.VMEM((16,), jnp.int32),
        pltpu.VMEM((NPT,), jnp.int32),
        pltpu.VMEM((PALL,), jnp.int32),
        pltpu.VMEM((PALL,), jnp.int32),
        pltpu.VMEM((NPT, 2 * H), jnp.float32),
        pltpu.VMEM((NPT, 2 * H), jnp.float32),
        pltpu.SemaphoreType.DMA,
        pltpu.SemaphoreType.DMA,
        pltpu.SemaphoreType.DMA,
        pltpu.SemaphoreType.DMA,
    ],
)
def _sc_gather(t0_hbm, st_hbm, srt_hbm, x1_hbm, seq_hbm,
               t0_v, st_v, p_v, sid_v, rows0_v, rows1_v,
               semg0, semg1, semw0, semw1):
    w = _wid()
    base = w * NPT
    pltpu.sync_copy(t0_hbm, t0_v)
    pltpu.sync_copy(st_hbm.at[pl.ds(base, NPT)], st_v)
    t0 = t0_v[...][0]

    for j in range(TCH):
        def pos_body(v, carry, j=j):
            s16 = st_v[pl.ds(v * 16, 16)]
            p_v[pl.ds(j * NPT + v * 16, 16)] = jnp.minimum(
                s16 + jnp.full((16,), t0 + j), E - 1)
            return carry

        lax.fori_loop(0, NSV, pos_body, 0)

    pltpu.sync_copy(srt_hbm.at[p_v], sid_v)

    bufs = (rows0_v, rows1_v)
    gsems = (semg0, semg1)
    wsems = (semw0, semw1)
    pend_g = [None, None]
    pend_w = [None, None]

    def fire(j):
        b = j & 1
        pend_g[b] = pltpu.async_copy(
            x1_hbm.at[sid_v.at[pl.ds(j * NPT, NPT)]], bufs[b], gsems[b])

    fire(0)
    for j in range(TCH):
        b = j & 1
        nb_ = (j + 1) & 1
        if j + 1 < TCH:
            if pend_w[nb_] is not None:
                pend_w[nb_].wait()
                pend_w[nb_] = None
            fire(j + 1)
        pend_g[b].wait()
        pend_w[b] = pltpu.async_copy(
            bufs[b], seq_hbm.at[j, pl.ds(base, NPT)], wsems[b])
    for b in (0, 1):
        if pend_w[b] is not None:
            pend_w[b].wait()


def _tc_lstm_body(sc_ref, cnt_ref, seq_ref, h_ref, c_ref, w_ref, b_ref,
                  ho_ref, co_ref):
    ho_ref[...] = h_ref[...]
    co_ref[...] = c_ref[...]
    rem = sc_ref[0]
    t0 = sc_ref[1]
    cnt = cnt_ref[...]
    for j in range(TCH):
        @pl.when(j < rem)
        def _():
            tf = (t0 + j).astype(jnp.float32)
            xt = jnp.where(cnt > tf, seq_ref[j][:, :H], 0.0)
            hh = ho_ref[...]
            xh = jnp.concatenate([xt, hh], axis=1)
            g = jax.lax.dot_general(xh, w_ref[...], (((1,), (0,)), ((), ())),
                                    precision=_HI) + b_ref[...]
            gi = g[:, 0 * H:1 * H]
            gf = g[:, 1 * H:2 * H]
            gg = g[:, 2 * H:3 * H]
            go = g[:, 3 * H:4 * H]
            cn = jax.nn.sigmoid(gf) * co_ref[...] + jax.nn.sigmoid(gi) * jnp.tanh(gg)
            co_ref[...] = cn
            ho_ref[...] = jax.nn.sigmoid(go) * jnp.tanh(cn)


def _tc_lstm(scal, cntf, seq, h, c, wcat, bias):
    nb = 8
    blk = NP // nb
    return pl.pallas_call(
        _tc_lstm_body,
        grid=(nb,),
        in_specs=[
            pl.BlockSpec(memory_space=pltpu.SMEM),
            pl.BlockSpec((blk, 1), lambda i: (i, 0)),
            pl.BlockSpec((TCH, blk, 2 * H), lambda i: (0, i, 0)),
            pl.BlockSpec((blk, H), lambda i: (i, 0)),
            pl.BlockSpec((blk, H), lambda i: (i, 0)),
            pl.BlockSpec((2 * H, 4 * H), lambda i: (0, 0)),
            pl.BlockSpec((1, 4 * H), lambda i: (0, 0)),
        ],
        out_specs=[
            pl.BlockSpec((blk, H), lambda i: (i, 0)),
            pl.BlockSpec((blk, H), lambda i: (i, 0)),
        ],
        out_shape=[
            jax.ShapeDtypeStruct((NP, H), jnp.float32),
            jax.ShapeDtypeStruct((NP, H), jnp.float32),
        ],
    )(scal, cntf, seq, h, c, wcat, bias)


def _tc_out_body(h_ref, x1_ref, wl_ref, wr_ref, b2_ref, o_ref):
    o = (jax.lax.dot_general(h_ref[...], wl_ref[...], (((1,), (0,)), ((), ())),
                             precision=_HI)
         + jax.lax.dot_general(x1_ref[...][:, :H], wr_ref[...], (((1,), (0,)), ((), ())),
                               precision=_HI)
         + b2_ref[...])
    m = jnp.max(o, axis=1, keepdims=True)
    z = o - m
    o_ref[...] = z - jnp.log(jnp.sum(jnp.exp(z), axis=1, keepdims=True))


def _tc_out(hT, x1p, wlT, wrT, b2):
    nb = 8
    blk = NP // nb
    return pl.pallas_call(
        _tc_out_body,
        grid=(nb,),
        in_specs=[
            pl.BlockSpec((blk, H), lambda i: (i, 0)),
            pl.BlockSpec((blk, 2 * H), lambda i: (i, 0)),
            pl.BlockSpec((H, H), lambda i: (0, 0)),
            pl.BlockSpec((H, H), lambda i: (0, 0)),
            pl.BlockSpec((1, H), lambda i: (0, 0)),
        ],
        out_specs=pl.BlockSpec((blk, H), lambda i: (i, 0)),
        out_shape=jax.ShapeDtypeStruct((NP, H), jnp.float32),
    )(hT, x1p, wlT, wrT, b2)


@jax.jit
def kernel(x, edge_index, W1_l, b1_l, W1_r, W_ih, W_hh, b_ih, b_hh, W2_l, b2_l, W2_r):
    src = edge_index[0]
    dst = edge_index[1]
    xp = jnp.pad(x, ((0, NP - N), (0, 0)))

    cnt_all = _sc_hist(dst)
    counts2, starts2, invc2, off3, maxd = _tc_offsets(cnt_all.reshape(NW, 80, 128))
    ylp, xr = _tc_proj(xp, W1_l.T, W1_r.T, b1_l.reshape(1, H))

    srt, spar = _sc_csr(dst, src, off3.reshape(NW, NP), ylp)
    x1p = _tc_x1(spar, invc2.reshape(NP, 1), xr)

    counts_f = counts2.reshape(NP)
    starts_f = starts2.reshape(NP)
    maxdeg = maxd[0, 0]

    wcat = jnp.concatenate([W_ih.T, W_hh.T], axis=0)
    bias = (b_ih + b_hh).reshape(1, 4 * H)

    h0 = jnp.zeros((NP, H), jnp.float32)
    c0 = jnp.zeros((NP, H), jnp.float32)

    def cond(carry):
        t0, _, _ = carry
        return t0 < maxdeg

    cnt_colf = counts_f.reshape(NP, 1).astype(jnp.float32)

    def body(carry):
        t0, h, c = carry
        t0a = jnp.full((16,), t0, jnp.int32)
        seq = _sc_gather(t0a, starts_f, srt, x1p)
        scal = jnp.stack([maxdeg - t0, t0])
        h, c = _tc_lstm(scal, cnt_colf, seq, h, c, wcat, bias)
        return t0 + TCH, h, c

    _, hT, _ = lax.while_loop(cond, body, (jnp.int32(0), h0, c0))

    out = _tc_out(hT, x1p, W2_l.T, W2_r.T, b2_l.reshape(1, H))
    return out[:N]


# speedup vs baseline: 1.0101x; 1.0091x over previous
import functools

import jax
import jax.numpy as jnp
from jax import lax
from jax.experimental import pallas as pl
from jax.experimental.pallas import tpu as pltpu
from jax.experimental.pallas import tpu_sc as plsc

N = 10000
E = 160000
F_IN = 256
H = 64
NP = 10240
NW = 32
EPW = E // NW
NPT = NP // NW
GSZ = 64
EPWP = 5120
NG = EPWP // GSZ
TCH = 10
ROWS_PER_SUB = NP // 16

_mesh = plsc.VectorSubcoreMesh(core_axis_name="c", subcore_axis_name="s")

_SC_PARAMS = pltpu.CompilerParams(needs_layout_passes=False)

_HI = jax.lax.Precision.HIGHEST


def _wid():
    return lax.axis_index("s") * 2 + lax.axis_index("c")


def _iota16():
    return lax.broadcasted_iota(jnp.int32, (16,), 0)


def _dup_ranks(d16):
    iota = _iota16()
    rank = jnp.zeros((16,), jnp.int32)
    later = jnp.zeros((16,), jnp.int32)
    for k in range(16):
        eq = d16 == jnp.full((16,), d16[k])
        rank = rank + jnp.where(eq & (iota > k), 1, 0)
        later = later + jnp.where(eq & (iota < k), 1, 0)
    return rank, later


@functools.partial(
    pl.kernel,
    out_type=jax.ShapeDtypeStruct((NW, NP), jnp.int32),
    mesh=_mesh,
    compiler_params=_SC_PARAMS,
    scratch_types=[
        pltpu.VMEM((EPWP,), jnp.int32),
        pltpu.VMEM((NP,), jnp.int32),
    ],
)
def _sc_hist(dst_hbm, out_hbm, dst_v, cnt_v):
    w = _wid()

    def pad_body(i, carry):
        dst_v[pl.ds(EPWP - 128 + i * 16, 16)] = jnp.full((16,), NP - 1, jnp.int32)
        return carry

    lax.fori_loop(0, 8, pad_body, 0)
    pltpu.sync_copy(dst_hbm.at[pl.ds(w * EPW, EPW)], dst_v.at[pl.ds(0, EPW)])

    def zero_body(i, carry):
        cnt_v[pl.ds(i * 16, 16)] = jnp.zeros((16,), jnp.int32)
        return carry

    lax.fori_loop(0, NP // 16, zero_body, 0)

    def hist_body(g, carry):
        d16 = dst_v[pl.ds(g * 16, 16)]
        rank, later = _dup_ranks(d16)
        cur = plsc.load_gather(cnt_v, [d16])
        valid = (_iota16() + jnp.full((16,), g * 16)) < EPW
        plsc.store_scatter(cnt_v, [d16], cur + rank + 1,
                           mask=valid & (later == 0))
        return carry

    lax.fori_loop(0, (EPW + 15) // 16, hist_body, 0)
    pltpu.sync_copy(cnt_v, out_hbm.at[w])


def _tc_offsets_body(cnt_ref, counts_ref, starts_ref, invc_ref, off_ref, maxd_ref):
    cnt = cnt_ref[...].astype(jnp.float32)
    counts = jnp.sum(cnt, axis=0)

    r0 = lax.broadcasted_iota(jnp.int32, (80, 80), 0)
    c0 = lax.broadcasted_iota(jnp.int32, (80, 80), 1)
    lt80 = (c0 < r0).astype(jnp.float32)
    r1 = lax.broadcasted_iota(jnp.int32, (128, 128), 0)
    c1 = lax.broadcasted_iota(jnp.int32, (128, 128), 1)
    ut128 = (r1 < c1).astype(jnp.float32)

    s_rows = jax.lax.dot_general(lt80, counts, (((1,), (0,)), ((), ())),
                                 precision=_HI)
    term1 = jnp.sum(s_rows, axis=1, keepdims=True)
    term2 = jax.lax.dot_general(counts, ut128, (((1,), (0,)), ((), ())),
                                precision=_HI)
    starts = term1 + term2

    counts_ref[...] = counts.astype(jnp.int32)
    starts_ref[...] = starts.astype(jnp.int32)
    invc_ref[...] = 1.0 / jnp.maximum(counts, 1.0)
    maxd_ref[...] = jnp.full((1, 1), jnp.max(counts), jnp.float32).astype(jnp.int32)

    acc = jnp.zeros((80, 128), jnp.float32)
    for w in range(NW):
        off_ref[w] = (acc + starts).astype(jnp.int32)
        acc = acc + cnt[w]


def _tc_offsets(cnt3):
    return pl.pallas_call(
        _tc_offsets_body,
        out_shape=[
            jax.ShapeDtypeStruct((80, 128), jnp.int32),
            jax.ShapeDtypeStruct((80, 128), jnp.int32),
            jax.ShapeDtypeStruct((80, 128), jnp.float32),
            jax.ShapeDtypeStruct((NW, 80, 128), jnp.int32),
            jax.ShapeDtypeStruct((1, 1), jnp.int32),
        ],
    )(cnt3)


def _tc_proj_body(x_ref, wl_ref, wr_ref, b1_ref, yl_ref, xr_ref):
    xb = x_ref[...]
    yl = jax.lax.dot_general(xb, wl_ref[...], (((1,), (0,)), ((), ())),
                             precision=_HI)
    yl_ref[...] = jnp.concatenate([yl, jnp.zeros_like(yl)], axis=1)
    xr_ref[...] = jax.lax.dot_general(xb, wr_ref[...], (((1,), (0,)), ((), ())),
                                      precision=_HI) + b1_ref[...]


def _tc_proj(xp, wlT, wrT, b1):
    nb = 8
    blk = NP // nb
    return pl.pallas_call(
        _tc_proj_body,
        grid=(nb,),
        in_specs=[
            pl.BlockSpec((blk, F_IN), lambda i: (i, 0)),
            pl.BlockSpec((F_IN, H), lambda i: (0, 0)),
            pl.BlockSpec((F_IN, H), lambda i: (0, 0)),
            pl.BlockSpec((1, H), lambda i: (0, 0)),
        ],
        out_specs=[
            pl.BlockSpec((blk, 2 * H), lambda i: (i, 0)),
            pl.BlockSpec((blk, H), lambda i: (i, 0)),
        ],
        out_shape=[
            jax.ShapeDtypeStruct((NP, 2 * H), jnp.float32),
            jax.ShapeDtypeStruct((NP, H), jnp.float32),
        ],
    )(xp, wlT, wrT, b1)


@functools.partial(
    pl.kernel,
    out_type=[
        jax.ShapeDtypeStruct((E + 2 * NW,), jnp.int32),
        jax.ShapeDtypeStruct((2, NP, 2 * H), jnp.float32),
    ],
    mesh=_mesh,
    compiler_params=_SC_PARAMS,
    scratch_types=[
        pltpu.VMEM((EPWP,), jnp.int32),
        pltpu.VMEM((EPWP,), jnp.int32),
        pltpu.VMEM((NP,), jnp.int32),
        pltpu.VMEM((NG, GSZ), jnp.int32),
        pltpu.VMEM((GSZ, 2 * H), jnp.float32),
        pltpu.VMEM((GSZ, 2 * H), jnp.float32),
        pltpu.VMEM_SHARED((NP, 2 * H), jnp.float32),
        pltpu.SemaphoreType.DMA,
        pltpu.SemaphoreType.DMA,
        pltpu.SemaphoreType.DMA,
        pltpu.SemaphoreType.DMA,
        pltpu.SemaphoreType.DMA,
    ],
)
def _sc_csr(dst_hbm, src_hbm, off_hbm, yl_hbm, srt_hbm, spar_hbm,
            src_v, dst_v, off_v, slot_v,
            rows0_v, rows1_v, s_sh,
            sem_s, semg0, semg1, sema0, sema1):
    c = lax.axis_index("c")
    s = lax.axis_index("s")
    w = _wid()

    def pad_body(i, carry):
        src_v[pl.ds(EPWP - 128 + i * 16, 16)] = jnp.full((16,), NP - 1, jnp.int32)
        dst_v[pl.ds(EPWP - 128 + i * 16, 16)] = jnp.full((16,), NP - 1, jnp.int32)
        return carry

    lax.fori_loop(0, 8, pad_body, 0)
    pltpu.sync_copy(src_hbm.at[pl.ds(w * EPW, EPW)], src_v.at[pl.ds(0, EPW)])
    pltpu.sync_copy(dst_hbm.at[pl.ds(w * EPW, EPW)], dst_v.at[pl.ds(0, EPW)])
    pltpu.sync_copy(off_hbm.at[w], off_v)

    def pre_body(i, carry):
        slot_v[lax.shift_right_logical(i, 2), pl.ds((i & 3) * 16, 16)] = (
            jnp.full((16,), E + 2 * w, jnp.int32))
        return carry

    lax.fori_loop(0, NG * 4, pre_body, 0)

    def zrow_body(i, carry):
        rows0_v[lax.shift_right_logical(i, 2), pl.ds((i & 3) * 16, 16)] = (
            jnp.zeros((16,), jnp.float32))
        return carry

    lax.fori_loop(0, GSZ * 4, zrow_body, 0)
    zc = [pltpu.async_copy(rows0_v,
                           s_sh.at[pl.ds(s * ROWS_PER_SUB + q * GSZ, GSZ)],
                           sem_s)
          for q in range(ROWS_PER_SUB // GSZ)]
    for cp in zc:
        cp.wait()
    plsc.subcore_barrier()

    def slot_body(i, carry):
        d16 = dst_v[pl.ds(i * 16, 16)]
        rank, later = _dup_ranks(d16)
        cur = plsc.load_gather(off_v, [d16])
        valid = (_iota16() + jnp.full((16,), i * 16)) < EPW
        plsc.store_scatter(off_v, [d16], cur + rank + 1,
                           mask=valid & (later == 0))
        g = lax.shift_right_logical(i, 2)
        r = lax.bitwise_and(i, 3) * 16
        slot_v[g, pl.ds(r, 16)] = cur + rank
        return carry

    lax.fori_loop(0, (EPW + 15) // 16, slot_body, 0)

    prev_wave = []
    for g0 in range(0, NG, 10):
        wave = [pltpu.async_copy(src_v.at[pl.ds(g * GSZ, GSZ)],
                                 srt_hbm.at[slot_v.at[g]], sem_s)
                for g in range(g0, min(g0 + 10, NG))]
        for cp in prev_wave:
            cp.wait()
        prev_wave = wave
    for cp in prev_wave:
        cp.wait()

    bufs = (rows0_v, rows1_v)
    gsems = (semg0, semg1)
    asems = (sema0, sema1)
    pend_g = [None, None]
    pend_a = [None, None]

    def fireg(g):
        b = g & 1
        pend_g[b] = pltpu.async_copy(
            yl_hbm.at[src_v.at[pl.ds(g * GSZ, GSZ)]], bufs[b], gsems[b])

    fireg(0)
    for g in range(NG):
        b = g & 1
        nb_ = (g + 1) & 1
        if g + 1 < NG:
            if pend_a[nb_] is not None:
                pend_a[nb_].wait()
                pend_a[nb_] = None
            fireg(g + 1)
        pend_g[b].wait()
        pend_a[b] = pltpu.async_copy(
            bufs[b], s_sh.at[dst_v.at[pl.ds(g * GSZ, GSZ)]],
            asems[b], add=True)
    for b in (0, 1):
        if pend_a[b] is not None:
            pend_a[b].wait()
    plsc.subcore_barrier()

    base = s * ROWS_PER_SUB
    pltpu.sync_copy(s_sh.at[pl.ds(base, ROWS_PER_SUB)],
                    spar_hbm.at[c, pl.ds(base, ROWS_PER_SUB)])


def _tc_x1_body(s_ref, invc_ref, xr_ref, x1_ref):
    i = pl.program_id(0)
    blk = x1_ref.shape[0]
    srow = (s_ref[0, :, :H] + s_ref[1, :, :H]) * invc_ref[...]
    h1 = srow + xr_ref[...]
    x1 = jnp.where(h1 > 0, h1, jnp.exp(jnp.minimum(h1, 0.0)) - 1.0)
    rows = i * blk + lax.broadcasted_iota(jnp.int32, (blk, 1), 0)
    x1 = jnp.where(rows < N, x1, 0.0)
    x1_ref[...] = jnp.concatenate([x1, jnp.zeros_like(x1)], axis=1)


def _tc_x1(spar, invc_col, xr):
    nb = 8
    blk = NP // nb
    return pl.pallas_call(
        _tc_x1_body,
        grid=(nb,),
        in_specs=[
            pl.BlockSpec((2, blk, 2 * H), lambda i: (0, i, 0)),
            pl.BlockSpec((blk, 1), lambda i: (i, 0)),
            pl.BlockSpec((blk, H), lambda i: (i, 0)),
        ],
        out_specs=pl.BlockSpec((blk, 2 * H), lambda i: (i, 0)),
        out_shape=jax.ShapeDtypeStruct((NP, 2 * H), jnp.float32),
    )(spar, invc_col, xr)


NSV = NPT // 16
PALL = TCH * NPT


@functools.partial(
    pl.kernel,
    out_type=jax.ShapeDtypeStruct((TCH, NP, 2 * H), jnp.float32),
    mesh=_mesh,
    compiler_params=_SC_PARAMS,
    scratch_types=[
        pltpu.VMEM((16,), jnp.int32),
        pltpu.VMEM((NPT,), jnp.int32),
        pltpu.VMEM((PALL,), jnp.int32),
        pltpu.VMEM((PALL,), jnp.int32),
        pltpu.VMEM((NPT, 2 * H), jnp.float32),
        pltpu.VMEM((NPT, 2 * H), jnp.float32),
        pltpu.SemaphoreType.DMA,
        pltpu.SemaphoreType.DMA,
        pltpu.SemaphoreType.DMA,
        pltpu.SemaphoreType.DMA,
    ],
)
def _sc_gather(t0_hbm, st_hbm, srt_hbm, x1_hbm, seq_hbm,
               t0_v, st_v, p_v, sid_v, rows0_v, rows1_v,
               semg0, semg1, semw0, semw1):
    w = _wid()
    base = w * NPT
    pltpu.sync_copy(t0_hbm, t0_v)
    pltpu.sync_copy(st_hbm.at[pl.ds(base, NPT)], st_v)
    t0 = t0_v[...][0]

    for j in range(TCH):
        def pos_body(v, carry, j=j):
            s16 = st_v[pl.ds(v * 16, 16)]
            p_v[pl.ds(j * NPT + v * 16, 16)] = jnp.minimum(
                s16 + jnp.full((16,), t0 + j), E - 1)
            return carry

        lax.fori_loop(0, NSV, pos_body, 0)

    pltpu.sync_copy(srt_hbm.at[p_v], sid_v)

    bufs = (rows0_v, rows1_v)
    gsems = (semg0, semg1)
    wsems = (semw0, semw1)
    pend_g = [None, None]
    pend_w = [None, None]

    def fire(j):
        b = j & 1
        pend_g[b] = pltpu.async_copy(
            x1_hbm.at[sid_v.at[pl.ds(j * NPT, NPT)]], bufs[b], gsems[b])

    fire(0)
    for j in range(TCH):
        b = j & 1
        nb_ = (j + 1) & 1
        if j + 1 < TCH:
            if pend_w[nb_] is not None:
                pend_w[nb_].wait()
                pend_w[nb_] = None
            fire(j + 1)
        pend_g[b].wait()
        pend_w[b] = pltpu.async_copy(
            bufs[b], seq_hbm.at[j, pl.ds(base, NPT)], wsems[b])
    for b in (0, 1):
        if pend_w[b] is not None:
            pend_w[b].wait()


def _tc_lstm_body(sc_ref, cnt_ref, seq_ref, h_ref, c_ref, w_ref, b_ref,
                  ho_ref, co_ref):
    ho_ref[...] = h_ref[...]
    co_ref[...] = c_ref[...]
    rem = sc_ref[0]
    t0 = sc_ref[1]
    cnt = cnt_ref[...]
    for j in range(TCH):
        @pl.when(j < rem)
        def _():
            tf = (t0 + j).astype(jnp.float32)
            xt = jnp.where(cnt > tf, seq_ref[j][:, :H], 0.0)
            hh = ho_ref[...]
            xh = jnp.concatenate([xt, hh], axis=1)
            g = jax.lax.dot_general(xh, w_ref[...], (((1,), (0,)), ((), ())),
                                    precision=_HI) + b_ref[...]
            gi = g[:, 0 * H:1 * H]
            gf = g[:, 1 * H:2 * H]
            gg = g[:, 2 * H:3 * H]
            go = g[:, 3 * H:4 * H]
            cn = jax.nn.sigmoid(gf) * co_ref[...] + jax.nn.sigmoid(gi) * jnp.tanh(gg)
            co_ref[...] = cn
            ho_ref[...] = jax.nn.sigmoid(go) * jnp.tanh(cn)


def _tc_lstm(scal, cntf, seq, h, c, wcat, bias):
    nb = 8
    blk = NP // nb
    return pl.pallas_call(
        _tc_lstm_body,
        grid=(nb,),
        in_specs=[
            pl.BlockSpec(memory_space=pltpu.SMEM),
            pl.BlockSpec((blk, 1), lambda i: (i, 0)),
            pl.BlockSpec((TCH, blk, 2 * H), lambda i: (0, i, 0)),
            pl.BlockSpec((blk, H), lambda i: (i, 0)),
            pl.BlockSpec((blk, H), lambda i: (i, 0)),
            pl.BlockSpec((2 * H, 4 * H), lambda i: (0, 0)),
            pl.BlockSpec((1, 4 * H), lambda i: (0, 0)),
        ],
        out_specs=[
            pl.BlockSpec((blk, H), lambda i: (i, 0)),
            pl.BlockSpec((blk, H), lambda i: (i, 0)),
        ],
        out_shape=[
            jax.ShapeDtypeStruct((NP, H), jnp.float32),
            jax.ShapeDtypeStruct((NP, H), jnp.float32),
        ],
    )(scal, cntf, seq, h, c, wcat, bias)


def _tc_out_body(h_ref, x1_ref, wl_ref, wr_ref, b2_ref, o_ref):
    o = (jax.lax.dot_general(h_ref[...], wl_ref[...], (((1,), (0,)), ((), ())),
                             precision=_HI)
         + jax.lax.dot_general(x1_ref[...][:, :H], wr_ref[...], (((1,), (0,)), ((), ())),
                               precision=_HI)
         + b2_ref[...])
    m = jnp.max(o, axis=1, keepdims=True)
    z = o - m
    o_ref[...] = z - jnp.log(jnp.sum(jnp.exp(z), axis=1, keepdims=True))


def _tc_out(hT, x1p, wlT, wrT, b2):
    nb = 8
    blk = NP // nb
    return pl.pallas_call(
        _tc_out_body,
        grid=(nb,),
        in_specs=[
            pl.BlockSpec((blk, H), lambda i: (i, 0)),
            pl.BlockSpec((blk, 2 * H), lambda i: (i, 0)),
            pl.BlockSpec((H, H), lambda i: (0, 0)),
            pl.BlockSpec((H, H), lambda i: (0, 0)),
            pl.BlockSpec((1, H), lambda i: (0, 0)),
        ],
        out_specs=pl.BlockSpec((blk, H), lambda i: (i, 0)),
        out_shape=jax.ShapeDtypeStruct((NP, H), jnp.float32),
    )(hT, x1p, wlT, wrT, b2)


@jax.jit
def kernel(x, edge_index, W1_l, b1_l, W1_r, W_ih, W_hh, b_ih, b_hh, W2_l, b2_l, W2_r):
    src = edge_index[0]
    dst = edge_index[1]
    xp = jnp.pad(x, ((0, NP - N), (0, 0)))

    cnt_all = _sc_hist(dst)
    counts2, starts2, invc2, off3, maxd = _tc_offsets(cnt_all.reshape(NW, 80, 128))
    ylp, xr = _tc_proj(xp, W1_l.T, W1_r.T, b1_l.reshape(1, H))

    srt, spar = _sc_csr(dst, src, off3.reshape(NW, NP), ylp)
    x1p = _tc_x1(spar, invc2.reshape(NP, 1), xr)

    counts_f = counts2.reshape(NP)
    starts_f = starts2.reshape(NP)
    maxdeg = maxd[0, 0]

    wcat = jnp.concatenate([W_ih.T, W_hh.T], axis=0)
    bias = (b_ih + b_hh).reshape(1, 4 * H)

    h0 = jnp.zeros((NP, H), jnp.float32)
    c0 = jnp.zeros((NP, H), jnp.float32)

    def cond(carry):
        t0, _, _ = carry
        return t0 < maxdeg

    cnt_colf = counts_f.reshape(NP, 1).astype(jnp.float32)

    def body(carry):
        t0, h, c = carry
        t0a = jnp.full((16,), t0, jnp.int32)
        seq = _sc_gather(t0a, starts_f, srt, x1p)
        scal = jnp.stack([maxdeg - t0, t0])
        h, c = _tc_lstm(scal, cnt_colf, seq, h, c, wcat, bias)
        return t0 + TCH, h, c

    _, hT, _ = lax.while_loop(cond, body, (jnp.int32(0), h0, c0))

    out = _tc_out(hT, x1p, W2_l.T, W2_r.T, b2_l.reshape(1, H))
    return out[:N]
